# baseline (speedup 1.0000x reference)
import jax
import jax.numpy as jnp
from jax import lax
from jax.experimental import pallas as pl
from jax.experimental.pallas import tpu as pltpu
from jax.experimental.pallas import tpu_sc as plsc

B, K, H, W = 1024, 3, 8, 8
HW = H * W
D = 128
NC, NS, L = 2, 16, 16
NW = NC * NS
BPW = B // NW
KHW = K * HW
NBUF = 8
DG0 = 4
DADD = 2


QUAD = 2


def _sc_body(inp_hbm, table_hbm, out_hbm, idx_v, idxT_v, accR, *sems):
    gsems = sems[:NBUF]
    osems = sems[NBUF:]
    accs = [accR.at[pl.ds(p * HW, HW)] for p in range(NBUF)]
    wid = lax.axis_index("s") * NC + lax.axis_index("c")
    b0 = wid * BPW

    pltpu.sync_copy(inp_hbm.at[:, pl.ds((wid // 4) * 128, 128)], idx_v)

    lanes = lax.iota(jnp.int32, L)
    sub = (wid % 4) * BPW

    @plsc.parallel_loop(0, KHW, 1, unroll=2)
    def _(r):
        rv = jnp.full((L,), r, dtype=jnp.int32)
        plsc.store_scatter(idxT_v, [lanes, rv], idx_v[r, pl.ds(sub, L)])
        plsc.store_scatter(idxT_v, [L + lanes, rv],
                           idx_v[r, pl.ds(sub + L, L)])

    def fire_g0(j, p):
        pltpu.async_copy(table_hbm.at[idxT_v.at[j, pl.ds(0, HW)]],
                         accs[p], gsems[p])

    def fire_adds(j, p):
        for k in range(1, K):
            pltpu.async_copy(table_hbm.at[idxT_v.at[j, pl.ds(k * HW, HW)]],
                             accs[p], gsems[p], add=True)

    def wait_g(p, n):
        for _ in range(n):
            pltpu.make_async_copy(table_hbm.at[idxT_v.at[0, pl.ds(0, HW)]],
                                  accs[p], gsems[p]).wait()

    def fire_quad(j, p):
        pltpu.async_copy(accR.at[pl.ds((p - QUAD + 1) * HW, QUAD * HW)],
                         out_hbm.at[pl.ds((b0 + j - QUAD + 1) * HW,
                                          QUAD * HW)],
                         osems[(p - QUAD + 1) // QUAD])

    def wait_quad(q):
        pltpu.make_async_copy(accR.at[pl.ds(0, QUAD * HW)],
                              out_hbm.at[pl.ds(0, QUAD * HW)],
                              osems[q]).wait()

    for j in range(DG0):
        fire_g0(j, j)
    for j in range(DADD):
        wait_g(j, 1)
        fire_adds(j, j)

    def pipe(t, c2):
        for p in range(NBUF):
            j = t * NBUF + p
            jj = j + DG0
            pj = (p + DG0) % NBUF

            @pl.when(jj < BPW)
            def _():
                @pl.when(jj >= NBUF)
                def _():
                    if pj % QUAD == 0:
                        wait_quad(pj // QUAD)
                fire_g0(jj, pj)

            ja = j + DADD
            pa = (p + DADD) % NBUF

            @pl.when(ja < BPW)
            def _():
                wait_g(pa, 1)
                fire_adds(ja, pa)

            wait_g(p, 2)
            if p % QUAD == QUAD - 1:
                fire_quad(j, p)
        return c2

    lax.fori_loop(0, BPW // NBUF, pipe, 0)
    for q in range(NBUF // QUAD):
        wait_quad(q)


def kernel(inputs, table):
    inp2 = inputs.transpose(1, 2, 3, 0).reshape(KHW, B)
    mesh = plsc.VectorSubcoreMesh(
        core_axis_name="c", subcore_axis_name="s",
        num_cores=NC, num_subcores=NS,
    )
    scratch = (
        [pltpu.VMEM((KHW, 128), jnp.int32),
         pltpu.VMEM((BPW, 256), jnp.int32)]
        + [pltpu.VMEM((NBUF * HW, D), jnp.float32)]
        + [pltpu.SemaphoreType.DMA for _ in range(NBUF + NBUF // QUAD)]
    )
    out = pl.kernel(
        _sc_body,
        out_type=jax.ShapeDtypeStruct((B * HW, D), jnp.float32),
        mesh=mesh,
        scratch_types=scratch,
        compiler_params=pltpu.CompilerParams(needs_layout_passes=False),
    )(inp2, table)
    return out.reshape(B, H, W, D).transpose(0, 3, 1, 2)
